# baseline (speedup 1.0000x reference)
import functools

import jax
import jax.numpy as jnp
from jax import lax
from jax.experimental import pallas as pl
from jax.experimental.pallas import tpu as pltpu
from jax.experimental.pallas import tpu_sc as plsc

B, N = 8, 12500
M = B * N
E = 500000
CH = 512
NCHUNK = 196
MP = NCHUNK * CH
KB = 128
EP = E + KB
NB_PAD = 224
NW = 32
BM = 2048
GRID = MP // BM
EPS = 1e-5


def _row_mask(i):
    rows = lax.broadcasted_iota(jnp.int32, (BM, 1), 0) + i * BM
    return rows < M


def _bn_from_stats(st):
    mean = st[0:1, :] * (1.0 / M)
    var = st[1:2, :] * (1.0 / M) - mean * mean
    inv = lax.rsqrt(jnp.maximum(var, 0.0) + EPS)
    return mean, inv


def _accum_stats(st_ref, z, i):
    zm = jnp.where(_row_mask(i), z, 0.0)
    s = jnp.concatenate(
        [jnp.sum(zm, axis=0, keepdims=True), jnp.sum(zm * zm, axis=0, keepdims=True)], axis=0
    )

    @pl.when(i == 0)
    def _():
        st_ref[...] = s

    @pl.when(i != 0)
    def _():
        st_ref[...] += s


def _mm_stats_body(x_ref, w_ref, z_ref, st_ref):
    i = pl.program_id(0)
    z = jnp.dot(x_ref[...], w_ref[...], preferred_element_type=jnp.float32)
    z_ref[...] = z
    _accum_stats(st_ref, z, i)


def _bn_mm_stats_body(z_ref, stin_ref, w_ref, zo_ref, sto_ref):
    i = pl.program_id(0)
    mean, inv = _bn_from_stats(stin_ref[...])
    a = jnp.maximum((z_ref[...] - mean) * inv, 0.0)
    z = jnp.dot(a, w_ref[...], preferred_element_type=jnp.float32)
    zo_ref[...] = z
    _accum_stats(sto_ref, z, i)


def _bn_scale_body(z_ref, stin_ref, dego_ref, h_ref):
    mean, inv = _bn_from_stats(stin_ref[...])
    a = jnp.maximum((z_ref[...] - mean) * inv, 0.0)
    do = dego_ref[...]
    ns = jnp.where(do > 0, lax.rsqrt(jnp.maximum(do, 1.0)), 0.0)
    h_ref[...] = a * ns


def _graph_mm_body(agg_ref, degi_ref, dego_ref, w_ref, b_ref, h_ref):
    di = degi_ref[...]
    nd = jnp.where(di > 0, lax.rsqrt(jnp.maximum(di, 1.0)), 0.0)
    g = jnp.dot(agg_ref[...] * nd, w_ref[...], preferred_element_type=jnp.float32)
    g = jnp.maximum(g + b_ref[...], 0.0)
    do = dego_ref[...]
    ns = jnp.where(do > 0, lax.rsqrt(jnp.maximum(do, 1.0)), 0.0)
    h_ref[...] = g * ns


def _g3_fc1_body(agg_ref, degi_ref, w3_ref, b3_ref, w4_ref, z_ref, st_ref):
    i = pl.program_id(0)
    di = degi_ref[...]
    nd = jnp.where(di > 0, lax.rsqrt(jnp.maximum(di, 1.0)), 0.0)
    g = jnp.dot(agg_ref[...] * nd, w3_ref[...], preferred_element_type=jnp.float32)
    g = g + b3_ref[...]
    z = jnp.dot(g, w4_ref[...], preferred_element_type=jnp.float32)
    z_ref[...] = z
    _accum_stats(st_ref, z, i)


def _head_body(z_ref, stin_ref, w_ref, b_ref, out_ref, acc_ref):
    i = pl.program_id(0)
    mean, inv = _bn_from_stats(stin_ref[...])
    a = jnp.maximum((z_ref[...] - mean) * inv, 0.0)
    a = jnp.where(_row_mask(i), a, -jnp.inf)
    pm = jnp.max(a, axis=0, keepdims=True)

    @pl.when(i == 0)
    def _():
        acc_ref[...] = pm

    @pl.when(i != 0)
    def _():
        acc_ref[...] = jnp.maximum(acc_ref[...], pm)

    @pl.when(i == GRID - 1)
    def _():
        out_ref[...] = (
            jnp.dot(acc_ref[...], w_ref[...], preferred_element_type=jnp.float32) + b_ref[...]
        )


def _rows_spec(c):
    return pl.BlockSpec((BM, c), lambda i: (i, 0))


def _full_spec(shape):
    return pl.BlockSpec(shape, lambda i: tuple(0 for _ in shape))


def _mm_stats(x, w):
    cin, cout = w.shape
    return pl.pallas_call(
        _mm_stats_body,
        grid=(GRID,),
        in_specs=[_rows_spec(cin), _full_spec(w.shape)],
        out_specs=[_rows_spec(cout), _full_spec((2, cout))],
        out_shape=[
            jax.ShapeDtypeStruct((MP, cout), jnp.float32),
            jax.ShapeDtypeStruct((2, cout), jnp.float32),
        ],
    )(x, w)


def _bn_mm_stats(z, st, w):
    cin, cout = w.shape
    return pl.pallas_call(
        _bn_mm_stats_body,
        grid=(GRID,),
        in_specs=[_rows_spec(cin), _full_spec((2, cin)), _full_spec(w.shape)],
        out_specs=[_rows_spec(cout), _full_spec((2, cout))],
        out_shape=[
            jax.ShapeDtypeStruct((MP, cout), jnp.float32),
            jax.ShapeDtypeStruct((2, cout), jnp.float32),
        ],
    )(z, st, w)


def _bn_scale(z, st, dego):
    c = z.shape[1]
    return pl.pallas_call(
        _bn_scale_body,
        grid=(GRID,),
        in_specs=[_rows_spec(c), _full_spec((2, c)), _rows_spec(1)],
        out_specs=_rows_spec(c),
        out_shape=jax.ShapeDtypeStruct((MP, c), jnp.float32),
    )(z, st, dego)


def _graph_mm(agg, degi, dego, w, b):
    cin, cout = w.shape
    return pl.pallas_call(
        _graph_mm_body,
        grid=(GRID,),
        in_specs=[
            _rows_spec(cin),
            _rows_spec(1),
            _rows_spec(1),
            _full_spec(w.shape),
            _full_spec((1, cout)),
        ],
        out_specs=_rows_spec(cout),
        out_shape=jax.ShapeDtypeStruct((MP, cout), jnp.float32),
    )(agg, degi, dego, w, b)


def _g3_fc1(agg, degi, w3, b3, w4):
    cout = w4.shape[1]
    return pl.pallas_call(
        _g3_fc1_body,
        grid=(GRID,),
        in_specs=[
            _rows_spec(w3.shape[0]),
            _rows_spec(1),
            _full_spec(w3.shape),
            _full_spec((1, w3.shape[1])),
            _full_spec(w4.shape),
        ],
        out_specs=[_rows_spec(cout), _full_spec((2, cout))],
        out_shape=[
            jax.ShapeDtypeStruct((MP, cout), jnp.float32),
            jax.ShapeDtypeStruct((2, cout), jnp.float32),
        ],
    )(agg, degi, w3, b3, w4)


def _head(z, st, w, b):
    cin = z.shape[1]
    cout = w.shape[1]
    return pl.pallas_call(
        _head_body,
        grid=(GRID,),
        in_specs=[
            _rows_spec(cin),
            _full_spec((2, cin)),
            _full_spec(w.shape),
            _full_spec((1, cout)),
        ],
        out_specs=_full_spec((1, cout)),
        out_shape=jax.ShapeDtypeStruct((1, cout), jnp.float32),
        scratch_shapes=[pltpu.VMEM((1, cin), jnp.float32)],
    )(z, st, w, b)


def _sc_mesh():
    return plsc.VectorSubcoreMesh(
        core_axis_name="c", subcore_axis_name="s", num_cores=2, num_subcores=16
    )


def _wid():
    return lax.axis_index("s") * 2 + lax.axis_index("c")


EB = (E + KB - 1) // KB
ESP = EB * KB
CAP = E + NCHUNK * NW * 16
CAPP = CAP + KB
SENTMIN = M << 9
HIST_W = 256


def _sent_vec():
    return ((M + 1 + lax.iota(jnp.int32, 16)) << 9) | 511


def _deg_body(srcraw_hbm, partials_hbm, vals_v, hist_v):
    wid = _wid()
    one16 = jnp.where(lax.iota(jnp.int32, 16) == 0, 1.0, 0.0)

    def hzero_body(r, _):
        hist_v[pl.ds(r * 16, 16)] = jnp.zeros((16,), jnp.float32)
        return 0

    lax.fori_loop(0, (MP + 16) // 16, hzero_body, 0)
    nblk = (EB - wid + NW - 1) // NW

    def hblk_body(t, _):
        b = wid + t * NW
        pltpu.sync_copy(srcraw_hbm.at[pl.ds(b * KB, KB)], vals_v.at[pl.ds(0, KB)])

        def hedge_body(e, _):
            ld = vals_v[pl.ds(e, 16)][0]
            plsc.addupdate(hist_v.at[pl.ds(ld, 16)], one16)
            return 0

        lax.fori_loop(0, KB, hedge_body, 0)
        return 0

    lax.fori_loop(0, nblk, hblk_body, 0)
    pltpu.sync_copy(hist_v.at[pl.ds(0, MP)], partials_hbm.at[wid])


def _b1_body(key_hbm, hist_hbm, vals_v, cnt_v):
    wid = _wid()
    one16i = jnp.where(lax.iota(jnp.int32, 16) == 0, 1, 0).astype(jnp.int32)

    def zero_body(r, _):
        cnt_v[pl.ds(r * 16, 16)] = jnp.zeros((16,), jnp.int32)
        return 0

    lax.fori_loop(0, (HIST_W + 16) // 16, zero_body, 0)
    nblk = (EB - wid + NW - 1) // NW

    def blk_body(t, _):
        b = wid + t * NW
        bstart = b * KB
        pltpu.sync_copy(key_hbm.at[pl.ds(bstart, KB)], vals_v.at[pl.ds(0, KB)])
        hi = jnp.minimum(E - bstart, KB)

        def edge_body(e, _):
            k = vals_v[pl.ds(e, 16)][0]
            plsc.addupdate(cnt_v.at[pl.ds(k, 16)], one16i)
            return 0

        lax.fori_loop(0, hi, edge_body, 0)
        return 0

    lax.fori_loop(0, nblk, blk_body, 0)
    pltpu.sync_copy(cnt_v.at[pl.ds(0, HIST_W)], hist_hbm.at[wid])


def _b2_body(key_hbm, pk_hbm, offs_hbm, tot_hbm, out_hbm, cur_v, key_v, pk_v, tot_v, stage):
    wid = _wid()
    one16i = jnp.where(lax.iota(jnp.int32, 16) == 0, 1, 0).astype(jnp.int32)
    lanes = lax.iota(jnp.int32, 16)
    sent = _sent_vec()
    pltpu.sync_copy(offs_hbm.at[wid], cur_v.at[pl.ds(0, HIST_W)])
    pltpu.sync_copy(tot_hbm, tot_v)

    def sinit_body(k, _):
        stage[k, :] = sent
        return 0

    lax.fori_loop(0, HIST_W, sinit_body, 0)
    nblk = (EB - wid + NW - 1) // NW

    def blk_body(t, _):
        b = wid + t * NW
        bstart = b * KB
        pltpu.sync_copy(key_hbm.at[pl.ds(bstart, KB)], key_v.at[pl.ds(0, KB)])
        pltpu.sync_copy(pk_hbm.at[pl.ds(bstart, KB)], pk_v.at[pl.ds(0, KB)])
        hi = jnp.minimum(E - bstart, KB)

        def edge_body(e, _):
            k = key_v[pl.ds(e, 16)][0]
            pos = cur_v[pl.ds(k, 16)][0]
            plsc.addupdate(cur_v.at[pl.ds(k, 16)], one16i)
            lane = pos & 15
            v = pk_v[pl.ds(e, 16)][0]
            stage[k, :] = jnp.where(lanes == lane, v, stage[k, :])

            @pl.when(lane == 15)
            def _():
                pltpu.sync_copy(
                    stage.at[k], out_hbm.at[pl.ds(pl.multiple_of(pos - 15, 16), 16)]
                )
                stage[k, :] = sent

            return 0

        lax.fori_loop(0, hi, edge_body, 0)
        return 0

    lax.fori_loop(0, nblk, blk_body, 0)

    def flush_body(k, _):
        pos = cur_v[pl.ds(k, 16)][0]
        lane = pos & 15

        @pl.when(lane != 0)
        def _():
            pltpu.sync_copy(
                stage.at[k], out_hbm.at[pl.ds(pl.multiple_of(pos - lane, 16), 16)]
            )

        return 0

    lax.fori_loop(0, NCHUNK, flush_body, 0)
    t0 = tot_v[pl.ds(0, 16)][0]
    nfill = (CAPP - t0 - wid * 16 + 16 * NW - 1) // (16 * NW)

    def fill_body(t, _):
        p = pl.multiple_of(t0 + (wid + t * NW) * 16, 16)
        pltpu.sync_copy(stage.at[HIST_W - 1], out_hbm.at[pl.ds(p, 16)])
        return 0

    lax.fori_loop(0, nfill, fill_body, 0)


def _gc_body(h_hbm, pk_hbm, bnd_hbm, out_hbm, cnt_hbm, bv,
             idx0, idx1, pk0, pk1, rows0, rows1, acc, cntc, sem0, sem1):
    wid = _wid()
    pltpu.sync_copy(bnd_hbm, bv)
    nchunks = (NCHUNK - wid + NW - 1) // NW
    bufs = ((idx0, pk0, rows0, sem0), (idx1, pk1, rows1, sem1))
    one16 = jnp.where(lax.iota(jnp.int32, 16) == 0, 1.0, 0.0)

    def chunk_body(t, _):
        cidx = wid + t * NW
        base = cidx * CH
        e0 = pl.multiple_of(bv[pl.ds(cidx, 16)][0], 16)
        e1 = bv[pl.ds(cidx + 1, 16)][0]

        def zero_body(r, _):
            for j in range(8):
                acc[r, pl.ds(j * 16, 16)] = jnp.zeros((16,), jnp.float32)
            return 0

        lax.fori_loop(0, CH, zero_body, 0)

        def czero_body(r, _):
            cntc[pl.ds(r * 16, 16)] = jnp.zeros((16,), jnp.float32)
            return 0

        lax.fori_loop(0, (CH + 16) // 16, czero_body, 0)
        nb = (e1 - e0 + KB - 1) // KB

        def issue(p, g):
            iv, pv, rv, sm = bufs[p]
            bstart = pl.multiple_of(e0 + g * KB, 16)
            pltpu.sync_copy(pk_hbm.at[pl.ds(bstart, KB)], pv.at[pl.ds(0, KB)])
            for j in range(KB // 16):
                sl = pl.ds(j * 16, 16)
                iv[sl] = pv[sl] >> 9
            pltpu.async_copy(h_hbm.at[iv], rv, sm)

        @pl.when(nb > 0)
        def _():
            issue(0, 0)

        def blk_body(g, _):
            for p in (0, 1):

                @pl.when((g & 1) == p)
                def _():
                    iv, pv, rv, sm = bufs[p]

                    @pl.when(g + 1 < nb)
                    def _():
                        issue(1 - p, g + 1)

                    pltpu.make_async_copy(h_hbm.at[iv], rv, sm).wait()
                    bstart = e0 + g * KB
                    hi = jnp.minimum(e1 - bstart, KB)

                    def edge_body(e, _):
                        v = pv[pl.ds(e, 16)][0]

                        @pl.when(v < SENTMIN)
                        def _():
                            ld = v & 511
                            for j in range(8):
                                sl = pl.ds(j * 16, 16)
                                plsc.addupdate(acc.at[ld, sl], rv[e, sl])
                            plsc.addupdate(cntc.at[pl.ds(ld, 16)], one16)

                        return 0

                    lax.fori_loop(0, hi, edge_body, 0)

            return 0

        lax.fori_loop(0, nb, blk_body, 0)
        pltpu.sync_copy(acc, out_hbm.at[pl.ds(base, CH)])
        pltpu.sync_copy(cntc.at[pl.ds(0, CH)], cnt_hbm.at[pl.ds(base, CH)])
        return 0

    lax.fori_loop(0, nchunks, chunk_body, 0)


def _deg_kernel(srcraw):
    k = pl.kernel(
        _deg_body,
        out_type=jax.ShapeDtypeStruct((NW, MP), jnp.float32),
        mesh=_sc_mesh(),
        scratch_types=[
            pltpu.VMEM((KB + 16,), jnp.int32),
            pltpu.VMEM((MP + 16,), jnp.float32),
        ],
    )
    return k(srcraw)


def _bucket1(keyp):
    k = pl.kernel(
        _b1_body,
        out_type=jax.ShapeDtypeStruct((NW, HIST_W), jnp.int32),
        mesh=_sc_mesh(),
        scratch_types=[
            pltpu.VMEM((KB + 16,), jnp.int32),
            pltpu.VMEM((HIST_W + 16,), jnp.int32),
        ],
    )
    return k(keyp)


def _bucket2(keyp, pkp, offs, tot):
    k = pl.kernel(
        _b2_body,
        out_type=jax.ShapeDtypeStruct((CAPP,), jnp.int32),
        mesh=_sc_mesh(),
        scratch_types=[
            pltpu.VMEM((HIST_W + 16,), jnp.int32),
            pltpu.VMEM((KB + 16,), jnp.int32),
            pltpu.VMEM((KB + 16,), jnp.int32),
            pltpu.VMEM((16,), jnp.int32),
            pltpu.VMEM((HIST_W, 16), jnp.int32),
        ],
    )
    return k(keyp, pkp, offs, tot)


def _colsum_body(p_ref, o_ref):
    o_ref[...] = jnp.sum(p_ref[...], axis=0, keepdims=True)


def _colsum(partials):
    return pl.pallas_call(
        _colsum_body,
        grid=(GRID,),
        in_specs=[pl.BlockSpec((NW, BM), lambda i: (0, i))],
        out_specs=pl.BlockSpec((1, BM), lambda i: (0, i)),
        out_shape=jax.ShapeDtypeStruct((1, MP), jnp.float32),
    )(partials)


def _gconv(h, pk, bnd):
    k = pl.kernel(
        _gc_body,
        out_type=[
            jax.ShapeDtypeStruct((MP, 128), jnp.float32),
            jax.ShapeDtypeStruct((MP,), jnp.float32),
        ],
        mesh=_sc_mesh(),
        scratch_types=[
            pltpu.VMEM((NB_PAD,), jnp.int32),
            pltpu.VMEM((KB,), jnp.int32),
            pltpu.VMEM((KB,), jnp.int32),
            pltpu.VMEM((KB + 16,), jnp.int32),
            pltpu.VMEM((KB + 16,), jnp.int32),
            pltpu.VMEM((KB, 128), jnp.float32),
            pltpu.VMEM((KB, 128), jnp.float32),
            pltpu.VMEM((CH, 128), jnp.float32),
            pltpu.VMEM((CH + 16,), jnp.float32),
            pltpu.SemaphoreType.DMA,
            pltpu.SemaphoreType.DMA,
        ],
    )
    return k(h, pk, bnd)


def kernel(pointcloud, edge_index, W1, W2, W3, W4, gcW1, gcb1, gcW2, gcb2, gcW3, gcb3,
           fc1W, fc2W, fc3W, fc3b):
    x = pointcloud
    centroids = jnp.mean(x, axis=2)
    d = jnp.sum((x - centroids[:, :, None, :]) ** 2, axis=1)[..., 0]
    idx = jnp.argsort(d, axis=-1)
    perm = (jnp.arange(B, dtype=jnp.int32)[:, None] * N + idx.astype(jnp.int32)).reshape(M)
    src2 = jnp.take(perm, edge_index[0], axis=0)
    dst2 = jnp.take(perm, edge_index[1], axis=0)
    srcraw = jnp.concatenate([src2, jnp.full((ESP - E,), M, jnp.int32)])
    key = dst2 >> 9
    pk = (src2 << 9) | (dst2 & 511)
    pad_k = jnp.full((ESP - E,), NCHUNK, jnp.int32)
    keyp = jnp.concatenate([key, pad_k])
    pkp = jnp.concatenate([pk, jnp.full((ESP - E,), SENTMIN, jnp.int32)])
    hist = _bucket1(keyp)
    cnt_ct = hist[:, :NCHUNK].T
    sz = ((cnt_ct + 15) // 16) * 16
    flat = sz.reshape(-1)
    ends = jnp.cumsum(flat)
    starts = (ends - flat).astype(jnp.int32)
    total = ends[-1].astype(jnp.int32)
    offs_ct = starts.reshape(NCHUNK, NW)
    offs_tc = jnp.pad(offs_ct.T, ((0, 0), (0, HIST_W - NCHUNK)))
    tot_arr = jnp.zeros((16,), jnp.int32).at[0].set(total)
    pk_sorted = _bucket2(keyp, pkp, offs_tc, tot_arr)
    bd = jnp.concatenate(
        [offs_ct[:, 0], jnp.full((NB_PAD - NCHUNK,), 1, jnp.int32) * total]
    )

    xf = jnp.transpose(x[..., 0], (0, 2, 1)).reshape(M, 3)
    xf = jnp.pad(xf, ((0, MP - M), (0, 0)))

    partials = _deg_kernel(srcraw)
    dego = _colsum(partials).reshape(MP, 1)

    z1, st1 = _mm_stats(xf, W1)
    z2, st2 = _bn_mm_stats(z1, st1, W2)
    z3, st3 = _bn_mm_stats(z2, st2, W3)
    z4, st4 = _bn_mm_stats(z3, st3, W4)

    h1 = _bn_scale(z4, st4, dego)
    agg1, cnt1 = _gconv(h1, pk_sorted, bd)
    degi = cnt1.reshape(MP, 1)
    h2 = _graph_mm(agg1, degi, dego, gcW1, gcb1.reshape(1, -1))
    agg2, _ = _gconv(h2, pk_sorted, bd)
    h3 = _graph_mm(agg2, degi, dego, gcW2, gcb2.reshape(1, -1))
    agg3, _ = _gconv(h3, pk_sorted, bd)

    z5, st5 = _g3_fc1(agg3, degi, gcW3, gcb3.reshape(1, -1), fc1W)
    z6, st6 = _bn_mm_stats(z5, st5, fc2W)
    out = _head(z6, st6, fc3W, fc3b.reshape(1, -1))
    return out.reshape(128)


# speedup vs baseline: 1.2294x; 1.2294x over previous
import functools

import jax
import jax.numpy as jnp
from jax import lax
from jax.experimental import pallas as pl
from jax.experimental.pallas import tpu as pltpu
from jax.experimental.pallas import tpu_sc as plsc

B, N = 8, 12500
M = B * N
E = 500000
CH = 512
NCHUNK = 196
MP = NCHUNK * CH
KB = 128
EP = E + KB
NB_PAD = 224
NW = 32
BM = 2048
GRID = MP // BM
EPS = 1e-5


def _row_mask(i):
    rows = lax.broadcasted_iota(jnp.int32, (BM, 1), 0) + i * BM
    return rows < M


def _bn_from_stats(st):
    mean = st[0:1, :] * (1.0 / M)
    var = st[1:2, :] * (1.0 / M) - mean * mean
    inv = lax.rsqrt(jnp.maximum(var, 0.0) + EPS)
    return mean, inv


def _accum_stats(st_ref, z, i):
    zm = jnp.where(_row_mask(i), z, 0.0)
    s = jnp.concatenate(
        [jnp.sum(zm, axis=0, keepdims=True), jnp.sum(zm * zm, axis=0, keepdims=True)], axis=0
    )

    @pl.when(i == 0)
    def _():
        st_ref[...] = s

    @pl.when(i != 0)
    def _():
        st_ref[...] += s


def _mm_stats_body(x_ref, w_ref, z_ref, st_ref):
    i = pl.program_id(0)
    z = jnp.dot(x_ref[...], w_ref[...], preferred_element_type=jnp.float32)
    z_ref[...] = z
    _accum_stats(st_ref, z, i)


def _bn_mm_stats_body(z_ref, stin_ref, w_ref, zo_ref, sto_ref):
    i = pl.program_id(0)
    mean, inv = _bn_from_stats(stin_ref[...])
    a = jnp.maximum((z_ref[...] - mean) * inv, 0.0)
    z = jnp.dot(a, w_ref[...], preferred_element_type=jnp.float32)
    zo_ref[...] = z
    _accum_stats(sto_ref, z, i)


def _bn_scale_body(z_ref, stin_ref, dego_ref, h_ref):
    mean, inv = _bn_from_stats(stin_ref[...])
    a = jnp.maximum((z_ref[...] - mean) * inv, 0.0)
    do = dego_ref[...]
    ns = jnp.where(do > 0, lax.rsqrt(jnp.maximum(do, 1.0)), 0.0)
    h_ref[...] = a * ns


def _graph_mm_body(agg_ref, degi_ref, dego_ref, w_ref, b_ref, h_ref):
    di = degi_ref[...]
    nd = jnp.where(di > 0, lax.rsqrt(jnp.maximum(di, 1.0)), 0.0)
    g = jnp.dot(agg_ref[...] * nd, w_ref[...], preferred_element_type=jnp.float32)
    g = jnp.maximum(g + b_ref[...], 0.0)
    do = dego_ref[...]
    ns = jnp.where(do > 0, lax.rsqrt(jnp.maximum(do, 1.0)), 0.0)
    h_ref[...] = g * ns


def _g3_fc1_body(agg_ref, degi_ref, w3_ref, b3_ref, w4_ref, z_ref, st_ref):
    i = pl.program_id(0)
    di = degi_ref[...]
    nd = jnp.where(di > 0, lax.rsqrt(jnp.maximum(di, 1.0)), 0.0)
    g = jnp.dot(agg_ref[...] * nd, w3_ref[...], preferred_element_type=jnp.float32)
    g = g + b3_ref[...]
    z = jnp.dot(g, w4_ref[...], preferred_element_type=jnp.float32)
    z_ref[...] = z
    _accum_stats(st_ref, z, i)


def _head_body(z_ref, stin_ref, w_ref, b_ref, out_ref, acc_ref):
    i = pl.program_id(0)
    mean, inv = _bn_from_stats(stin_ref[...])
    a = jnp.maximum((z_ref[...] - mean) * inv, 0.0)
    a = jnp.where(_row_mask(i), a, -jnp.inf)
    pm = jnp.max(a, axis=0, keepdims=True)

    @pl.when(i == 0)
    def _():
        acc_ref[...] = pm

    @pl.when(i != 0)
    def _():
        acc_ref[...] = jnp.maximum(acc_ref[...], pm)

    @pl.when(i == GRID - 1)
    def _():
        out_ref[...] = (
            jnp.dot(acc_ref[...], w_ref[...], preferred_element_type=jnp.float32) + b_ref[...]
        )


def _rows_spec(c):
    return pl.BlockSpec((BM, c), lambda i: (i, 0))


def _full_spec(shape):
    return pl.BlockSpec(shape, lambda i: tuple(0 for _ in shape))


def _mm_stats(x, w):
    cin, cout = w.shape
    return pl.pallas_call(
        _mm_stats_body,
        grid=(GRID,),
        in_specs=[_rows_spec(cin), _full_spec(w.shape)],
        out_specs=[_rows_spec(cout), _full_spec((2, cout))],
        out_shape=[
            jax.ShapeDtypeStruct((MP, cout), jnp.float32),
            jax.ShapeDtypeStruct((2, cout), jnp.float32),
        ],
    )(x, w)


def _bn_mm_stats(z, st, w):
    cin, cout = w.shape
    return pl.pallas_call(
        _bn_mm_stats_body,
        grid=(GRID,),
        in_specs=[_rows_spec(cin), _full_spec((2, cin)), _full_spec(w.shape)],
        out_specs=[_rows_spec(cout), _full_spec((2, cout))],
        out_shape=[
            jax.ShapeDtypeStruct((MP, cout), jnp.float32),
            jax.ShapeDtypeStruct((2, cout), jnp.float32),
        ],
    )(z, st, w)


def _bn_scale(z, st, dego):
    c = z.shape[1]
    return pl.pallas_call(
        _bn_scale_body,
        grid=(GRID,),
        in_specs=[_rows_spec(c), _full_spec((2, c)), _rows_spec(1)],
        out_specs=_rows_spec(c),
        out_shape=jax.ShapeDtypeStruct((MP, c), jnp.float32),
    )(z, st, dego)


def _graph_mm(agg, degi, dego, w, b):
    cin, cout = w.shape
    return pl.pallas_call(
        _graph_mm_body,
        grid=(GRID,),
        in_specs=[
            _rows_spec(cin),
            _rows_spec(1),
            _rows_spec(1),
            _full_spec(w.shape),
            _full_spec((1, cout)),
        ],
        out_specs=_rows_spec(cout),
        out_shape=jax.ShapeDtypeStruct((MP, cout), jnp.float32),
    )(agg, degi, dego, w, b)


def _g3_fc1(agg, degi, w3, b3, w4):
    cout = w4.shape[1]
    return pl.pallas_call(
        _g3_fc1_body,
        grid=(GRID,),
        in_specs=[
            _rows_spec(w3.shape[0]),
            _rows_spec(1),
            _full_spec(w3.shape),
            _full_spec((1, w3.shape[1])),
            _full_spec(w4.shape),
        ],
        out_specs=[_rows_spec(cout), _full_spec((2, cout))],
        out_shape=[
            jax.ShapeDtypeStruct((MP, cout), jnp.float32),
            jax.ShapeDtypeStruct((2, cout), jnp.float32),
        ],
    )(agg, degi, w3, b3, w4)


def _head(z, st, w, b):
    cin = z.shape[1]
    cout = w.shape[1]
    return pl.pallas_call(
        _head_body,
        grid=(GRID,),
        in_specs=[
            _rows_spec(cin),
            _full_spec((2, cin)),
            _full_spec(w.shape),
            _full_spec((1, cout)),
        ],
        out_specs=_full_spec((1, cout)),
        out_shape=jax.ShapeDtypeStruct((1, cout), jnp.float32),
        scratch_shapes=[pltpu.VMEM((1, cin), jnp.float32)],
    )(z, st, w, b)


def _sc_mesh():
    return plsc.VectorSubcoreMesh(
        core_axis_name="c", subcore_axis_name="s", num_cores=2, num_subcores=16
    )


def _wid():
    return lax.axis_index("s") * 2 + lax.axis_index("c")


EB = (E + KB - 1) // KB
ESP = EB * KB


def _deg_body(srcraw_hbm, sdst_hbm, bd_hbm, partials_hbm, degi_hbm, bd_v, vals_v, cnt_v, hist_v):
    wid = _wid()
    pltpu.sync_copy(bd_hbm, bd_v)
    one16 = jnp.where(lax.iota(jnp.int32, 16) == 0, 1.0, 0.0)

    def hzero_body(r, _):
        hist_v[pl.ds(r * 16, 16)] = jnp.zeros((16,), jnp.float32)
        return 0

    lax.fori_loop(0, (MP + 16) // 16, hzero_body, 0)
    nblk = (EB - wid + NW - 1) // NW

    def hblk_body(t, _):
        b = wid + t * NW
        pltpu.sync_copy(srcraw_hbm.at[pl.ds(b * KB, KB)], vals_v.at[pl.ds(0, KB)])

        def hedge_body(e, _):
            ld = vals_v[pl.ds(e, 16)][0]
            plsc.addupdate(hist_v.at[pl.ds(ld, 16)], one16)
            return 0

        lax.fori_loop(0, KB, hedge_body, 0)
        return 0

    lax.fori_loop(0, nblk, hblk_body, 0)
    pltpu.sync_copy(hist_v.at[pl.ds(0, MP)], partials_hbm.at[wid])

    nchunks = (NCHUNK - wid + NW - 1) // NW

    def chunk_body(t, _):
        cidx = wid + t * NW
        base = cidx * CH
        e0 = bd_v[pl.ds(cidx, 16)][0]
        e1 = bd_v[pl.ds(cidx + 1, 16)][0]

        def zero_body(r, _):
            cnt_v[pl.ds(r * 16, 16)] = jnp.zeros((16,), jnp.float32)
            return 0

        lax.fori_loop(0, (CH + 16) // 16, zero_body, 0)
        e0a = (e0 // 8) * 8
        nb = (e1 - e0a + KB - 1) // KB

        def blk_body(g, _):
            bstart = e0a + g * KB
            pltpu.sync_copy(sdst_hbm.at[pl.ds(bstart, KB)], vals_v.at[pl.ds(0, KB)])
            lo = jnp.maximum(e0 - bstart, 0)
            hi = jnp.minimum(e1 - bstart, KB)

            def edge_body(e, _):
                ld = vals_v[pl.ds(e, 16)][0] - base
                plsc.addupdate(cnt_v.at[pl.ds(ld, 16)], one16)
                return 0

            lax.fori_loop(lo, hi, edge_body, 0)
            return 0

        lax.fori_loop(0, nb, blk_body, 0)
        pltpu.sync_copy(cnt_v.at[pl.ds(0, CH)], degi_hbm.at[pl.ds(base, CH)])
        return 0

    lax.fori_loop(0, nchunks, chunk_body, 0)


def _gc_body(h_hbm, src_hbm, dst_hbm, bnd_hbm, out_hbm, bv,
             idx0, idx1, dst0, dst1, rows0, rows1, acc, sem0, sem1):
    wid = _wid()
    pltpu.sync_copy(bnd_hbm, bv)
    nchunks = (NCHUNK - wid + NW - 1) // NW
    bufs = ((idx0, dst0, rows0, sem0), (idx1, dst1, rows1, sem1))

    def chunk_body(t, _):
        cidx = wid + t * NW
        base = cidx * CH
        e0 = bv[pl.ds(cidx, 16)][0]
        e1 = bv[pl.ds(cidx + 1, 16)][0]

        def zero_body(r, _):
            for j in range(8):
                acc[r, pl.ds(j * 16, 16)] = jnp.zeros((16,), jnp.float32)
            return 0

        lax.fori_loop(0, CH, zero_body, 0)
        e0a = (e0 // 8) * 8
        nb = (e1 - e0a + KB - 1) // KB

        def issue(p, g):
            iv, dv, rv, sm = bufs[p]
            bstart = e0a + g * KB
            pltpu.sync_copy(src_hbm.at[pl.ds(bstart, KB)], iv)
            pltpu.async_copy(h_hbm.at[iv], rv, sm)
            pltpu.sync_copy(dst_hbm.at[pl.ds(bstart, KB)], dv.at[pl.ds(0, KB)])

        @pl.when(nb > 0)
        def _():
            issue(0, 0)

        def blk_body(g, _):
            for p in (0, 1):

                @pl.when((g & 1) == p)
                def _():
                    iv, dv, rv, sm = bufs[p]

                    @pl.when(g + 1 < nb)
                    def _():
                        issue(1 - p, g + 1)

                    pltpu.make_async_copy(h_hbm.at[iv], rv, sm).wait()
                    bstart = e0a + g * KB
                    lo = jnp.maximum(e0 - bstart, 0)
                    hi = jnp.minimum(e1 - bstart, KB)

                    def edge_body(e, _):
                        ld = dv[pl.ds(e, 16)][0] - base
                        for j in range(8):
                            sl = pl.ds(j * 16, 16)
                            plsc.addupdate(acc.at[ld, sl], rv[e, sl])
                        return 0

                    lax.fori_loop(lo, hi, edge_body, 0)

            return 0

        lax.fori_loop(0, nb, blk_body, 0)
        pltpu.sync_copy(acc, out_hbm.at[pl.ds(base, CH)])
        return 0

    lax.fori_loop(0, nchunks, chunk_body, 0)


def _deg_kernel(srcraw, sdst, bd):
    k = pl.kernel(
        _deg_body,
        out_type=[
            jax.ShapeDtypeStruct((NW, MP), jnp.float32),
            jax.ShapeDtypeStruct((MP,), jnp.float32),
        ],
        mesh=_sc_mesh(),
        scratch_types=[
            pltpu.VMEM((NB_PAD,), jnp.int32),
            pltpu.VMEM((KB + 16,), jnp.int32),
            pltpu.VMEM((CH + 16,), jnp.float32),
            pltpu.VMEM((MP + 16,), jnp.float32),
        ],
    )
    return k(srcraw, sdst, bd)


def _colsum_body(p_ref, o_ref):
    o_ref[...] = jnp.sum(p_ref[...], axis=0, keepdims=True)


def _colsum(partials):
    return pl.pallas_call(
        _colsum_body,
        grid=(GRID,),
        in_specs=[pl.BlockSpec((NW, BM), lambda i: (0, i))],
        out_specs=pl.BlockSpec((1, BM), lambda i: (0, i)),
        out_shape=jax.ShapeDtypeStruct((1, MP), jnp.float32),
    )(partials)


def _gconv(h, src, dst, bnd):
    k = pl.kernel(
        _gc_body,
        out_type=jax.ShapeDtypeStruct((MP, 128), jnp.float32),
        mesh=_sc_mesh(),
        scratch_types=[
            pltpu.VMEM((NB_PAD,), jnp.int32),
            pltpu.VMEM((KB,), jnp.int32),
            pltpu.VMEM((KB,), jnp.int32),
            pltpu.VMEM((KB + 16,), jnp.int32),
            pltpu.VMEM((KB + 16,), jnp.int32),
            pltpu.VMEM((KB, 128), jnp.float32),
            pltpu.VMEM((KB, 128), jnp.float32),
            pltpu.VMEM((CH, 128), jnp.float32),
            pltpu.SemaphoreType.DMA,
            pltpu.SemaphoreType.DMA,
        ],
    )
    return k(h, src, dst, bnd)


def kernel(pointcloud, edge_index, W1, W2, W3, W4, gcW1, gcb1, gcW2, gcb2, gcW3, gcb3,
           fc1W, fc2W, fc3W, fc3b):
    x = pointcloud
    centroids = jnp.mean(x, axis=2)
    d = jnp.sum((x - centroids[:, :, None, :]) ** 2, axis=1)[..., 0]
    idx = jnp.argsort(d, axis=-1)
    perm = (jnp.arange(B, dtype=jnp.int32)[:, None] * N + idx.astype(jnp.int32)).reshape(M)
    src2 = jnp.take(perm, edge_index[0], axis=0)
    dst2 = jnp.take(perm, edge_index[1], axis=0)
    order = jnp.argsort(dst2)
    src_s = jnp.take(src2, order, axis=0)
    dst_s = jnp.take(dst2, order, axis=0)
    pad_i = jnp.zeros((KB,), jnp.int32)
    src_p = jnp.concatenate([src_s, pad_i])
    dst_p = jnp.concatenate([dst_s, pad_i])
    srcraw = jnp.concatenate([src2, jnp.full((ESP - E,), M, jnp.int32)])
    chunk_starts = jnp.arange(NCHUNK + 1, dtype=jnp.int32) * CH
    bd = jnp.searchsorted(dst_s, chunk_starts, side="left").astype(jnp.int32)
    bpad = jnp.full((NB_PAD - NCHUNK - 1,), E, jnp.int32)
    bd = jnp.concatenate([bd, bpad])

    xf = jnp.transpose(x[..., 0], (0, 2, 1)).reshape(M, 3)
    xf = jnp.pad(xf, ((0, MP - M), (0, 0)))

    partials, degi_v = _deg_kernel(srcraw, dst_p, bd)
    dego = _colsum(partials).reshape(MP, 1)
    degi = degi_v.reshape(MP, 1)

    z1, st1 = _mm_stats(xf, W1)
    z2, st2 = _bn_mm_stats(z1, st1, W2)
    z3, st3 = _bn_mm_stats(z2, st2, W3)
    z4, st4 = _bn_mm_stats(z3, st3, W4)

    h1 = _bn_scale(z4, st4, dego)
    agg1 = _gconv(h1, src_p, dst_p, bd)
    h2 = _graph_mm(agg1, degi, dego, gcW1, gcb1.reshape(1, -1))
    agg2 = _gconv(h2, src_p, dst_p, bd)
    h3 = _graph_mm(agg2, degi, dego, gcW2, gcb2.reshape(1, -1))
    agg3 = _gconv(h3, src_p, dst_p, bd)

    z5, st5 = _g3_fc1(agg3, degi, gcW3, gcb3.reshape(1, -1), fc1W)
    z6, st6 = _bn_mm_stats(z5, st5, fc2W)
    out = _head(z6, st6, fc3W, fc3b.reshape(1, -1))
    return out.reshape(128)
